# Initial kernel scaffold; baseline (speedup 1.0000x reference)
import jax
import jax.numpy as jnp
from jax.experimental import pallas as pl


def kernel(document_mention_indices, W, b):
    raise NotImplementedError("write your pallas kernel here")


# speedup vs baseline: 9.5272x; 9.5272x over previous
import functools

import jax
import jax.numpy as jnp
from jax import lax
from jax.experimental import pallas as pl
from jax.experimental.pallas import tpu as pltpu
from jax.experimental.pallas import tpu_sc as plsc

BATCH = 1024
MPD = 50
EMB = 128
LANES = 16
NC, NS = 2, 16
NW = NC * NS
DOCS_PER_W = BATCH // NW


def _sc_embedding_bag(idx, table, bias):
    mesh = plsc.VectorSubcoreMesh(core_axis_name="c", subcore_axis_name="s")

    @functools.partial(
        pl.kernel,
        out_type=jax.ShapeDtypeStruct((BATCH, EMB), jnp.float32),
        mesh=mesh,
        scratch_types=[
            pltpu.VMEM((DOCS_PER_W, MPD), jnp.int32),
            pltpu.VMEM((MPD, EMB), jnp.float32),
            pltpu.VMEM((DOCS_PER_W, EMB), jnp.float32),
            pltpu.VMEM((EMB,), jnp.float32),
        ],
    )
    def kern(idx_hbm, tab_hbm, b_hbm, out_hbm, idx_v, rows_v, out_v, bias_v):
        wid = lax.axis_index("s") * NC + lax.axis_index("c")
        base = wid * DOCS_PER_W
        pltpu.sync_copy(b_hbm, bias_v)
        pltpu.sync_copy(idx_hbm.at[pl.ds(base, DOCS_PER_W)], idx_v)

        @pl.loop(0, DOCS_PER_W)
        def _doc(d):
            pltpu.sync_copy(tab_hbm.at[idx_v.at[d]], rows_v)
            accs = [bias_v[pl.ds(c * LANES, LANES)] for c in range(EMB // LANES)]
            for r in range(MPD):
                for c in range(EMB // LANES):
                    accs[c] = accs[c] + rows_v[r, pl.ds(c * LANES, LANES)]
            for c in range(EMB // LANES):
                out_v[d, pl.ds(c * LANES, LANES)] = jnp.maximum(accs[c], 0.0)

        pltpu.sync_copy(out_v, out_hbm.at[pl.ds(base, DOCS_PER_W)])

    return kern(idx, table, bias)


def kernel(document_mention_indices, W, b):
    idx = document_mention_indices.astype(jnp.int32)
    table = W.T
    return _sc_embedding_bag(idx, table, b)


# speedup vs baseline: 10.9091x; 1.1450x over previous
import functools

import jax
import jax.numpy as jnp
from jax import lax
from jax.experimental import pallas as pl
from jax.experimental.pallas import tpu as pltpu
from jax.experimental.pallas import tpu_sc as plsc

BATCH = 1024
MPD = 50
EMB = 128
LANES = 16
NC, NS = 2, 16
NW = NC * NS
DOCS_PER_W = BATCH // NW


PAIR = 2
PAIRS_PER_W = DOCS_PER_W // PAIR
NBUF = 2


def _sc_embedding_bag(idx, table, bias):
    mesh = plsc.VectorSubcoreMesh(core_axis_name="c", subcore_axis_name="s")

    @functools.partial(
        pl.kernel,
        out_type=jax.ShapeDtypeStruct((BATCH, EMB), jnp.float32),
        mesh=mesh,
        scratch_types=[
            pltpu.VMEM((PAIRS_PER_W, PAIR * MPD), jnp.int32),
            pltpu.VMEM((NBUF, PAIR * MPD, EMB), jnp.float32),
            pltpu.VMEM((DOCS_PER_W, EMB), jnp.float32),
            pltpu.VMEM((EMB,), jnp.float32),
        ]
        + [pltpu.SemaphoreType.DMA] * NBUF,
    )
    def kern(idx_hbm, tab_hbm, b_hbm, out_hbm, idx_v, rows_v, out_v, bias_v,
             *sems):
        wid = lax.axis_index("s") * NC + lax.axis_index("c")
        base = wid * DOCS_PER_W
        pltpu.sync_copy(b_hbm, bias_v)
        pltpu.sync_copy(idx_hbm.at[pl.ds(wid * PAIRS_PER_W, PAIRS_PER_W)], idx_v)

        for j in range(NBUF):
            pltpu.async_copy(tab_hbm.at[idx_v.at[j]], rows_v.at[j], sems[j])

        @pl.loop(0, PAIRS_PER_W, step=NBUF)
        def _pair(p0):
            for j in range(NBUF):
                p = p0 + j
                pltpu.make_async_copy(
                    tab_hbm.at[idx_v.at[p]], rows_v.at[j], sems[j]).wait()
                for sub in range(PAIR):
                    accs = [bias_v[pl.ds(c * LANES, LANES)]
                            for c in range(EMB // LANES)]
                    for r in range(sub * MPD, (sub + 1) * MPD):
                        for c in range(EMB // LANES):
                            accs[c] = accs[c] + rows_v[j, r, pl.ds(c * LANES, LANES)]
                    d = p * PAIR + sub
                    for c in range(EMB // LANES):
                        out_v[d, pl.ds(c * LANES, LANES)] = jnp.maximum(accs[c], 0.0)

                @pl.when(p + NBUF < PAIRS_PER_W)
                def _():
                    pltpu.async_copy(
                        tab_hbm.at[idx_v.at[p + NBUF]], rows_v.at[j], sems[j])

        pltpu.sync_copy(out_v, out_hbm.at[pl.ds(base, DOCS_PER_W)])

    return kern(idx, table, bias)


def kernel(document_mention_indices, W, b):
    idx = document_mention_indices.astype(jnp.int32).reshape(
        BATCH // PAIR, PAIR * MPD)
    table = W.T
    return _sc_embedding_bag(idx, table, b)


# speedup vs baseline: 18.6823x; 1.7125x over previous
import functools

import jax
import jax.numpy as jnp
from jax import lax
from jax.experimental import pallas as pl
from jax.experimental.pallas import tpu as pltpu
from jax.experimental.pallas import tpu_sc as plsc

BATCH = 1024
MPD = 50
EMB = 128
LANES = 16
NC, NS = 2, 16
NW = NC * NS
DOCS_PER_W = BATCH // NW


PAIR = 2
PAIRS_PER_W = DOCS_PER_W // PAIR
NBUF = 2


def _sc_embedding_bag(idx, table, bias):
    mesh = plsc.VectorSubcoreMesh(core_axis_name="c", subcore_axis_name="s")

    @functools.partial(
        pl.kernel,
        out_type=jax.ShapeDtypeStruct((BATCH, EMB), jnp.float32),
        mesh=mesh,
        scratch_types=[
            pltpu.VMEM((PAIRS_PER_W, PAIR * MPD), jnp.int32),
        ]
        + [pltpu.VMEM((PAIR * MPD, EMB), jnp.float32)] * NBUF
        + [
            pltpu.VMEM((DOCS_PER_W, EMB), jnp.float32),
            pltpu.VMEM((EMB,), jnp.float32),
            pltpu.VMEM((EMB,), jnp.float32),
        ]
        + [pltpu.SemaphoreType.DMA] * NBUF,
    )
    def kern(idx_hbm, tab_hbm, b_hbm, out_hbm, idx_v, *rest):
        rows_bufs = rest[:NBUF]
        out_v, bias_v, acc_v = rest[NBUF], rest[NBUF + 1], rest[NBUF + 2]
        sems = rest[NBUF + 3:]
        wid = lax.axis_index("s") * NC + lax.axis_index("c")
        base = wid * DOCS_PER_W
        pltpu.sync_copy(b_hbm, bias_v)
        pltpu.sync_copy(idx_hbm.at[pl.ds(wid * PAIRS_PER_W, PAIRS_PER_W)], idx_v)

        for j in range(NBUF):
            pltpu.async_copy(tab_hbm.at[idx_v.at[j]], rows_bufs[j], sems[j])

        @pl.loop(0, PAIRS_PER_W, step=NBUF)
        def _pair(p0):
            for j in range(NBUF):
                p = p0 + j
                rows = rows_bufs[j]
                pltpu.make_async_copy(
                    tab_hbm.at[idx_v.at[p]], rows, sems[j]).wait()
                for sub in range(PAIR):
                    accs0 = tuple(bias_v[pl.ds(c * LANES, LANES)]
                                  for c in range(EMB // LANES))

                    def body(r, accs):
                        return tuple(
                            accs[c] + rows[r, pl.ds(c * LANES, LANES)]
                            for c in range(EMB // LANES))

                    accs = plsc.parallel_loop(
                        sub * MPD, (sub + 1) * MPD, 1, unroll=5,
                        carry=accs0)(body)
                    d = p * PAIR + sub
                    for c in range(EMB // LANES):
                        out_v[d, pl.ds(c * LANES, LANES)] = jnp.maximum(
                            accs[c], 0.0)

                @pl.when(p + NBUF < PAIRS_PER_W)
                def _():
                    pltpu.async_copy(
                        tab_hbm.at[idx_v.at[p + NBUF]], rows, sems[j])

        pltpu.sync_copy(out_v, out_hbm.at[pl.ds(base, DOCS_PER_W)])

    return kern(idx, table, bias)


def kernel(document_mention_indices, W, b):
    idx = document_mention_indices.astype(jnp.int32).reshape(
        BATCH // PAIR, PAIR * MPD)
    table = W.T
    return _sc_embedding_bag(idx, table, b)


# speedup vs baseline: 20.4062x; 1.0923x over previous
import functools

import jax
import jax.numpy as jnp
from jax import lax
from jax.experimental import pallas as pl
from jax.experimental.pallas import tpu as pltpu
from jax.experimental.pallas import tpu_sc as plsc

BATCH = 1024
MPD = 50
EMB = 128
LANES = 16
NC, NS = 2, 16
NW = NC * NS
DOCS_PER_W = BATCH // NW


PAIR = 2
PAIRS_PER_W = DOCS_PER_W // PAIR
NBUF = 4


def _sc_embedding_bag(idx, table, bias):
    mesh = plsc.VectorSubcoreMesh(core_axis_name="c", subcore_axis_name="s")

    @functools.partial(
        pl.kernel,
        out_type=jax.ShapeDtypeStruct((BATCH, EMB), jnp.float32),
        mesh=mesh,
        scratch_types=[
            pltpu.VMEM((PAIRS_PER_W, PAIR * MPD), jnp.int32),
        ]
        + [pltpu.VMEM((PAIR * MPD, EMB), jnp.float32)] * NBUF
        + [
            pltpu.VMEM((DOCS_PER_W, EMB), jnp.float32),
            pltpu.VMEM((EMB,), jnp.float32),
            pltpu.VMEM((EMB,), jnp.float32),
        ]
        + [pltpu.SemaphoreType.DMA] * NBUF,
    )
    def kern(idx_hbm, tab_hbm, b_hbm, out_hbm, idx_v, *rest):
        rows_bufs = rest[:NBUF]
        out_v, bias_v, acc_v = rest[NBUF], rest[NBUF + 1], rest[NBUF + 2]
        sems = rest[NBUF + 3:]
        wid = lax.axis_index("s") * NC + lax.axis_index("c")
        base = wid * DOCS_PER_W
        pltpu.sync_copy(b_hbm, bias_v)
        pltpu.sync_copy(idx_hbm.at[pl.ds(wid * PAIRS_PER_W, PAIRS_PER_W)], idx_v)

        for j in range(NBUF):
            pltpu.async_copy(tab_hbm.at[idx_v.at[j]], rows_bufs[j], sems[j])

        @pl.loop(0, PAIRS_PER_W, step=NBUF)
        def _pair(p0):
            for j in range(NBUF):
                p = p0 + j
                rows = rows_bufs[j]
                pltpu.make_async_copy(
                    tab_hbm.at[idx_v.at[p]], rows, sems[j]).wait()
                for sub in range(PAIR):
                    accs0 = tuple(bias_v[pl.ds(c * LANES, LANES)]
                                  for c in range(EMB // LANES))

                    def body(r, accs):
                        return tuple(
                            accs[c] + rows[r, pl.ds(c * LANES, LANES)]
                            for c in range(EMB // LANES))

                    accs = plsc.parallel_loop(
                        sub * MPD, (sub + 1) * MPD, 1, unroll=5,
                        carry=accs0)(body)
                    d = p * PAIR + sub
                    for c in range(EMB // LANES):
                        out_v[d, pl.ds(c * LANES, LANES)] = jnp.maximum(
                            accs[c], 0.0)

                @pl.when(p + NBUF < PAIRS_PER_W)
                def _():
                    pltpu.async_copy(
                        tab_hbm.at[idx_v.at[p + NBUF]], rows, sems[j])

        pltpu.sync_copy(out_v, out_hbm.at[pl.ds(base, DOCS_PER_W)])

    return kern(idx, table, bias)


def kernel(document_mention_indices, W, b):
    idx = document_mention_indices.astype(jnp.int32).reshape(
        BATCH // PAIR, PAIR * MPD)
    table = W.T
    return _sc_embedding_bag(idx, table, b)
